# baseline (speedup 1.0000x reference)
import functools

import jax
import jax.numpy as jnp
from jax import lax
from jax.experimental import pallas as pl
from jax.experimental.pallas import tpu as pltpu
from jax.experimental.pallas import tpu_sc as plsc

NC = 2
NS = 16
NW = NC * NS

D = 32
L = 128
GA = 10
GB = 5
PITCH = 33


def _build_gather(n_seq):
  n_groups = n_seq // GA
  assert n_groups % 2 == 0
  mesh = plsc.VectorSubcoreMesh(
      core_axis_name="c", subcore_axis_name="s",
      num_cores=NC, num_subcores=NS)

  @functools.partial(
      pl.kernel,
      out_type=jax.ShapeDtypeStruct((NW, n_seq, L, D), jnp.float32),
      mesh=mesh,
      scratch_types=[
          pltpu.VMEM((n_seq, L), jnp.int32),
          pltpu.VMEM((2, GA, L, D), jnp.float32),
          pltpu.SemaphoreType.DMA((2,)),
          pltpu.SemaphoreType.DMA((2,)),
      ],
      compiler_params=pltpu.CompilerParams(use_tc_tiling_on_sc=False),
  )
  def gather_k(tok_hbm, w_hbm, mid_hbm, idx_v, raw_v, gsem, wsem):
    wid = lax.axis_index("s") * NC + lax.axis_index("c")
    for st in range(n_seq // 8):
      pltpu.sync_copy(tok_hbm.at[st, wid], idx_v.at[pl.ds(st * 8, 8)])

    def gathers(g, b):
      for j in range(GA):
        yield pltpu.make_async_copy(
            w_hbm.at[idx_v.at[g * GA + j]], raw_v.at[b, j], gsem.at[b])

    def write(g, b):
      return pltpu.make_async_copy(
          raw_v.at[b], mid_hbm.at[wid].at[pl.ds(g * GA, GA)], wsem.at[b])

    def half(k, g, b, fire_next):
      for cp in gathers(g, b):
        cp.wait()
      fire_next()
      @pl.when(k > 0)
      def _():
        write(g, b).wait()
      write(g, b).start()

    def fire_group(g, b):
      for cp in gathers(g, b):
        cp.start()

    fire_group(0, 0)

    @pl.loop(0, n_groups // 2)
    def k_loop(k):
      g0 = 2 * k
      half(k, g0, 0, lambda: fire_group(g0 + 1, 1))

      def fire_even():
        @pl.when(k < n_groups // 2 - 1)
        def _():
          fire_group(g0 + 2, 0)
      half(k, g0 + 1, 1, fire_even)

    for b in range(2):
      write(n_groups - 2 + b, b).wait()

  return gather_k


def _build_transpose(n_seq):
  n_groups = n_seq // GB
  assert n_groups % 2 == 0
  mesh = plsc.VectorSubcoreMesh(
      core_axis_name="c", subcore_axis_name="s",
      num_cores=NC, num_subcores=NS)

  @functools.partial(
      pl.kernel,
      out_type=jax.ShapeDtypeStruct((n_seq, D // 8, NW, 8, L), jnp.float32),
      mesh=mesh,
      scratch_types=[
          pltpu.VMEM((2, GB, L, PITCH), jnp.float32),
          pltpu.VMEM((2, GB, D, L), jnp.float32),
          pltpu.SemaphoreType.DMA((2,)),
          pltpu.SemaphoreType.DMA((2,)),
      ],
      compiler_params=pltpu.CompilerParams(
          use_tc_tiling_on_sc=False, needs_layout_passes=False),
  )
  def transpose_k(mid_hbm, out_hbm, pad_v, tr_v, rsem, wsem):
    wid = lax.axis_index("s") * NC + lax.axis_index("c")
    iota = lax.broadcasted_iota(jnp.int32, (16,), 0)
    dvs = [jnp.full((16,), d, jnp.int32) for d in range(D)]

    def reads(g, b):
      yield pltpu.make_async_copy(
          mid_hbm.at[wid].at[pl.ds(g * GB, GB)],
          pad_v.at[b].at[:, :, pl.ds(0, D)], rsem.at[b])

    def writes(g, b):
      for dt in range(D // 8):
        yield pltpu.make_async_copy(
            tr_v.at[b].at[:, pl.ds(dt * 8, 8), :],
            out_hbm.at[pl.ds(g * GB, GB), dt, wid], wsem.at[b])

    def transpose(b):
      for u in range(GB):
        pad2 = pad_v.at[b, u]

        @plsc.parallel_loop(0, L, step=16, unroll=8)
        def t_loop(t0, u=u, pad2=pad2):
          tv = iota + t0
          for d in range(D):
            tr_v[b, u, d, pl.ds(t0, 16)] = plsc.load_gather(pad2, [tv, dvs[d]])

    def half(k, g, b, fire_next):
      for cp in reads(g, b):
        cp.wait()
      fire_next()
      @pl.when(k > 0)
      def _():
        for cp in writes(g, b):
          cp.wait()
      transpose(b)
      for cp in writes(g, b):
        cp.start()

    def fire_reads(g, b):
      for cp in reads(g, b):
        cp.start()

    fire_reads(0, 0)

    @pl.loop(0, n_groups // 2)
    def k_loop(k):
      g0 = 2 * k
      half(k, g0, 0, lambda: fire_reads(g0 + 1, 1))

      def fire_even():
        @pl.when(k < n_groups // 2 - 1)
        def _():
          fire_reads(g0 + 2, 0)
      half(k, g0 + 1, 1, fire_even)

    for b in range(2):
      for cp in writes(n_groups - 2 + b, b):
        cp.wait()

  return transpose_k


def kernel(token_ids, weight):
  bsz, seq = token_ids.shape
  assert bsz == NW * L
  tok4d = (token_ids.T.reshape(seq // 8, 8, NW, L)
           .transpose(0, 2, 1, 3))
  mid = _build_gather(seq)(tok4d, weight)
  out5d = _build_transpose(seq)(mid)
  return out5d.transpose(2, 4, 0, 1, 3).reshape(bsz, seq, D)


# speedup vs baseline: 1.1477x; 1.1477x over previous
import functools

import jax
import jax.numpy as jnp
from jax import lax
from jax.experimental import pallas as pl
from jax.experimental.pallas import tpu as pltpu
from jax.experimental.pallas import tpu_sc as plsc

NC = 2
NS = 16
NW = NC * NS

D = 32
L = 128
GA = 10
GB = 5
PITCH = 33


def _build_gather(n_seq):
  n_groups = n_seq // GA
  assert n_groups % 2 == 0
  mesh = plsc.VectorSubcoreMesh(
      core_axis_name="c", subcore_axis_name="s",
      num_cores=NC, num_subcores=NS)

  @functools.partial(
      pl.kernel,
      out_type=jax.ShapeDtypeStruct((NW, n_seq, L, D), jnp.float32),
      mesh=mesh,
      scratch_types=[
          pltpu.VMEM((n_seq, L), jnp.int32),
          pltpu.VMEM((2, GA, L, D), jnp.float32),
          pltpu.SemaphoreType.DMA((2,)),
          pltpu.SemaphoreType.DMA((2,)),
      ],
      compiler_params=pltpu.CompilerParams(use_tc_tiling_on_sc=False),
  )
  def gather_k(tok_hbm, w_hbm, mid_hbm, idx_v, raw_v, gsem, wsem):
    wid = lax.axis_index("s") * NC + lax.axis_index("c")
    for st in range(n_seq // 8):
      pltpu.sync_copy(tok_hbm.at[st, wid], idx_v.at[pl.ds(st * 8, 8)])

    def gathers(g, b):
      for j in range(GA):
        yield pltpu.make_async_copy(
            w_hbm.at[idx_v.at[g * GA + j]], raw_v.at[b, j], gsem.at[b])

    def write(g, b):
      return pltpu.make_async_copy(
          raw_v.at[b], mid_hbm.at[wid].at[pl.ds(g * GA, GA)], wsem.at[b])

    def half(k, g, b, fire_next):
      for cp in gathers(g, b):
        cp.wait()
      fire_next()
      @pl.when(k > 0)
      def _():
        write(g, b).wait()
      write(g, b).start()

    def fire_group(g, b):
      for cp in gathers(g, b):
        cp.start()

    fire_group(0, 0)

    @pl.loop(0, n_groups // 2)
    def k_loop(k):
      g0 = 2 * k
      half(k, g0, 0, lambda: fire_group(g0 + 1, 1))

      def fire_even():
        @pl.when(k < n_groups // 2 - 1)
        def _():
          fire_group(g0 + 2, 0)
      half(k, g0 + 1, 1, fire_even)

    for b in range(2):
      write(n_groups - 2 + b, b).wait()

  return gather_k


def _build_transpose(n_seq):
  n_groups = n_seq // GB
  assert n_groups % 2 == 0
  mesh = plsc.VectorSubcoreMesh(
      core_axis_name="c", subcore_axis_name="s",
      num_cores=NC, num_subcores=NS)

  @functools.partial(
      pl.kernel,
      out_type=jax.ShapeDtypeStruct((n_seq, D // 8, NW, 8, L), jnp.float32),
      mesh=mesh,
      scratch_types=[
          pltpu.VMEM((2, GB, L, PITCH), jnp.float32),
          pltpu.VMEM((2, GB, D, L), jnp.float32),
          pltpu.SemaphoreType.DMA((2,)),
          pltpu.SemaphoreType.DMA((2,)),
      ],
      compiler_params=pltpu.CompilerParams(
          use_tc_tiling_on_sc=False, needs_layout_passes=False),
  )
  def transpose_k(mid_hbm, out_hbm, pad_v, tr_v, rsem, wsem):
    wid = lax.axis_index("s") * NC + lax.axis_index("c")
    iota = lax.broadcasted_iota(jnp.int32, (16,), 0)
    dvs = [jnp.full((16,), d, jnp.int32) for d in range(D)]

    def reads(g, b):
      yield pltpu.make_async_copy(
          mid_hbm.at[wid].at[pl.ds(g * GB, GB)],
          pad_v.at[b].at[:, :, pl.ds(0, D)], rsem.at[b])

    def writes(g, b):
      for dt in range(D // 8):
        yield pltpu.make_async_copy(
            tr_v.at[b].at[:, pl.ds(dt * 8, 8), :],
            out_hbm.at[pl.ds(g * GB, GB), dt, wid], wsem.at[b])

    def transpose(b):
      for u in range(GB):
        pad2 = pad_v.at[b, u]

        @plsc.parallel_loop(0, L, step=16, unroll=4)
        def t_loop(t0, u=u, pad2=pad2):
          tv = iota + t0
          for d in range(D):
            tr_v[b, u, d, pl.ds(t0, 16)] = plsc.load_gather(pad2, [tv, dvs[d]])

    def half(k, g, b, fire_next):
      for cp in reads(g, b):
        cp.wait()
      fire_next()
      @pl.when(k > 0)
      def _():
        for cp in writes(g, b):
          cp.wait()
      transpose(b)
      for cp in writes(g, b):
        cp.start()

    def fire_reads(g, b):
      for cp in reads(g, b):
        cp.start()

    fire_reads(0, 0)

    @pl.loop(0, n_groups // 2)
    def k_loop(k):
      g0 = 2 * k
      half(k, g0, 0, lambda: fire_reads(g0 + 1, 1))

      def fire_even():
        @pl.when(k < n_groups // 2 - 1)
        def _():
          fire_reads(g0 + 2, 0)
      half(k, g0 + 1, 1, fire_even)

    for b in range(2):
      for cp in writes(n_groups - 2 + b, b):
        cp.wait()

  return transpose_k


def kernel(token_ids, weight):
  bsz, seq = token_ids.shape
  assert bsz == NW * L
  tok4d = (token_ids.T.reshape(seq // 8, 8, NW, L)
           .transpose(0, 2, 1, 3))
  mid = _build_gather(seq)(tok4d, weight)
  out5d = _build_transpose(seq)(mid)
  return out5d.transpose(2, 4, 0, 1, 3).reshape(bsz, seq, D)
